# Initial kernel scaffold; baseline (speedup 1.0000x reference)
import jax
import jax.numpy as jnp
from jax.experimental import pallas as pl


def kernel(parent_features, child_to_parent_idx):
    raise NotImplementedError("write your pallas kernel here")


# speedup vs baseline: 3.5802x; 3.5802x over previous
import functools

import jax
import jax.numpy as jnp
from jax import lax
from jax.experimental import pallas as pl
from jax.experimental.pallas import tpu as pltpu
from jax.experimental.pallas import tpu_sc as plsc

V = 10000
D = 128
B = 320000
NC, NS = 2, 16
NW = NC * NS
BPW = B // NW
CH = 80
NCH = BPW // CH

_mesh = plsc.VectorSubcoreMesh(core_axis_name="c", subcore_axis_name="s")


@functools.partial(
    pl.kernel,
    mesh=_mesh,
    out_type=jax.ShapeDtypeStruct((B, D), jnp.float32),
    scratch_types=[
        pltpu.VMEM((NCH, CH), jnp.int32),
        pltpu.VMEM((CH, D), jnp.float32),
        pltpu.SemaphoreType.DMA,
    ],
)
def _gather_kernel(table_hbm, idx_hbm, out_hbm, idx_v, buf, gsem):
    wid = lax.axis_index("s") * NC + lax.axis_index("c")
    base = wid * BPW
    pltpu.sync_copy(idx_hbm.at[wid], idx_v)

    def body(c, _):
        pltpu.async_copy(table_hbm.at[idx_v.at[c]], buf, gsem).wait()
        pltpu.sync_copy(buf, out_hbm.at[pl.ds(base + c * CH, CH)])
        return 0

    lax.fori_loop(0, NCH, body, 0)


def kernel(parent_features, child_to_parent_idx):
    idx3d = child_to_parent_idx.astype(jnp.int32).reshape(NW, NCH, CH)
    return _gather_kernel(parent_features, idx3d)


# speedup vs baseline: 6.0126x; 1.6794x over previous
import functools

import jax
import jax.numpy as jnp
from jax import lax
from jax.experimental import pallas as pl
from jax.experimental.pallas import tpu as pltpu
from jax.experimental.pallas import tpu_sc as plsc

V = 10000
D = 128
B = 320000
NC, NS = 2, 16
NW = NC * NS
BPW = B // NW
CH = 80
NCH = BPW // CH
G = 5
GR = G * CH
NG = NCH // G

_mesh = plsc.VectorSubcoreMesh(core_axis_name="c", subcore_axis_name="s")


@functools.partial(
    pl.kernel,
    mesh=_mesh,
    out_type=jax.ShapeDtypeStruct((B, D), jnp.float32),
    scratch_types=[
        pltpu.VMEM((NCH, CH), jnp.int32),
        pltpu.VMEM((2, GR, D), jnp.float32),
        pltpu.SemaphoreType.DMA,
        pltpu.SemaphoreType.DMA,
        pltpu.SemaphoreType.DMA,
        pltpu.SemaphoreType.DMA,
    ],
)
def _gather_kernel(table_hbm, idx_hbm, out_hbm, idx_v, bufs, g0, g1, o0, o1):
    wid = lax.axis_index("s") * NC + lax.axis_index("c")
    base = wid * BPW
    gsem = (g0, g1)
    osem = (o0, o1)
    pltpu.sync_copy(idx_hbm.at[wid], idx_v)

    def fire_group(g, p):
        for j in range(G):
            pltpu.async_copy(
                table_hbm.at[idx_v.at[g * G + j]],
                bufs.at[p, pl.ds(j * CH, CH)],
                gsem[p],
            )

    def drain_group(p):
        pltpu.make_async_copy(
            table_hbm.at[pl.ds(0, GR)], bufs.at[p], gsem[p]
        ).wait()

    def out_copy(g, p):
        pltpu.async_copy(bufs.at[p], out_hbm.at[pl.ds(base + g * GR, GR)], osem[p])

    def drain_out(p):
        pltpu.make_async_copy(
            bufs.at[p], out_hbm.at[pl.ds(base, GR)], osem[p]
        ).wait()

    fire_group(0, 0)

    def body(i, _):
        g0_ = 2 * i
        for p in (0, 1):
            g = g0_ + p
            @pl.when(g >= 1)
            def _():
                drain_out(1 - p)
            @pl.when(g < NG - 1)
            def _():
                fire_group(g + 1, 1 - p)
            @pl.when(g < NG)
            def _():
                drain_group(p)
            @pl.when(g < NG)
            def _():
                out_copy(g, p)
        return 0

    lax.fori_loop(0, (NG + 2) // 2, body, 0)


def kernel(parent_features, child_to_parent_idx):
    idx3d = child_to_parent_idx.astype(jnp.int32).reshape(NW, NCH, CH)
    return _gather_kernel(parent_features, idx3d)


# speedup vs baseline: 8.9767x; 1.4930x over previous
import functools

import jax
import jax.numpy as jnp
from jax import lax
from jax.experimental import pallas as pl
from jax.experimental.pallas import tpu as pltpu
from jax.experimental.pallas import tpu_sc as plsc

V = 10000
D = 128
B = 320000
NC, NS = 2, 16
NW = NC * NS
BPW = B // NW
CH = 80
NCH = BPW // CH
G = 1
GR = G * CH
NG = NCH // G

_mesh = plsc.VectorSubcoreMesh(core_axis_name="c", subcore_axis_name="s")


@functools.partial(
    pl.kernel,
    mesh=_mesh,
    out_type=jax.ShapeDtypeStruct((B, D), jnp.float32),
    scratch_types=[
        pltpu.VMEM((NCH, CH), jnp.int32),
        pltpu.VMEM((2, GR, D), jnp.float32),
        pltpu.VMEM_SHARED((V, D), jnp.float32),
        pltpu.SemaphoreType.DMA,
        pltpu.SemaphoreType.DMA,
        pltpu.SemaphoreType.DMA,
        pltpu.SemaphoreType.DMA,
    ],
)
def _gather_kernel(table_hbm, idx_hbm, out_hbm, idx_v, bufs, table_sp,
                   g0, g1, o0, o1):
    wid = lax.axis_index("s") * NC + lax.axis_index("c")
    base = wid * BPW
    gsem = (g0, g1)
    osem = (o0, o1)

    @pl.when(lax.axis_index("s") == 0)
    def _():
        pltpu.sync_copy(table_hbm, table_sp)

    pltpu.sync_copy(idx_hbm.at[wid], idx_v)
    plsc.subcore_barrier()

    def fire_group(g, p):
        for j in range(G):
            pltpu.async_copy(
                table_sp.at[idx_v.at[g * G + j]],
                bufs.at[p, pl.ds(j * CH, CH)],
                gsem[p],
            )

    def drain_group(p):
        pltpu.make_async_copy(
            table_hbm.at[pl.ds(0, GR)], bufs.at[p], gsem[p]
        ).wait()

    def out_copy(g, p):
        pltpu.async_copy(bufs.at[p], out_hbm.at[pl.ds(base + g * GR, GR)], osem[p])

    def drain_out(p):
        pltpu.make_async_copy(
            bufs.at[p], out_hbm.at[pl.ds(base, GR)], osem[p]
        ).wait()

    fire_group(0, 0)

    def body(i, _):
        g0_ = 2 * i
        for p in (0, 1):
            g = g0_ + p

            @pl.when(g >= 1)
            def _():
                drain_out(1 - p)

            @pl.when(g < NG - 1)
            def _():
                fire_group(g + 1, 1 - p)

            @pl.when(g < NG)
            def _():
                drain_group(p)

            @pl.when(g < NG)
            def _():
                out_copy(g, p)
        return 0

    lax.fori_loop(0, (NG + 2) // 2, body, 0)


def kernel(parent_features, child_to_parent_idx):
    idx3d = child_to_parent_idx.astype(jnp.int32).reshape(NW, NCH, CH)
    return _gather_kernel(parent_features, idx3d)
